# baseline (speedup 1.0000x reference)
import math

import jax
import jax.numpy as jnp
from jax import lax
from jax.experimental import pallas as pl
from jax.experimental.pallas import tpu as pltpu

TOPK = 8
B = 8
T = 60
N = 196
D = 512
H = 4
HD = D // H
NB = 12
LAG = 6


def _fused_kernel(q_ref, key_ref, wq_ref, wk_ref, bq_ref, bk_ref,
                  value_ref, out_ref, bufs, sin, sout):
    dn = (((1,), (1,)), ((), ()))
    Q_all = lax.dot_general(key_ref[...], wq_ref[...], dn,
                            preferred_element_type=jnp.float32,
                            precision=lax.Precision.HIGHEST) + bq_ref[...]
    scale = 1.0 / math.sqrt(HD)
    iota_t = lax.broadcasted_iota(jnp.int32, (T, 1), 0)

    frames = []
    for b in range(B):
        K_b = lax.dot_general(q_ref[b], wk_ref[...], dn,
                              preferred_element_type=jnp.float32,
                              precision=lax.Precision.HIGHEST) + bk_ref[...]
        KQ = K_b * Q_all[b:b + 1, :]
        tw = jnp.zeros((T, 1), jnp.float32)
        for h in range(H):
            s = jnp.sum(KQ[:, h * HD:(h + 1) * HD], axis=1, keepdims=True) * scale
            m = jnp.max(s, axis=0, keepdims=True)
            e = jnp.exp(s - m)
            tw = tw + e / jnp.sum(e, axis=0, keepdims=True)

        sel = iota_t < 0
        cur = tw
        for _ in range(TOPK):
            vmax = jnp.max(cur, axis=0, keepdims=True)
            cand = jnp.where(cur >= vmax, iota_t, -1)
            pick = jnp.max(cand, axis=0, keepdims=True)
            picked = iota_t == pick
            sel = sel | picked
            cur = jnp.where(picked, -jnp.inf, cur)

        mask = sel
        for k in range(TOPK):
            t_k = jnp.min(jnp.where(mask, iota_t, T + 1))
            mask = mask & (iota_t != t_k)
            frames.append((b, k, t_k))

    ins = [None] * B
    outs = [None] * B

    def start_out(bj):
        for c in ins[bj]:
            c.wait()
        o = pltpu.make_async_copy(bufs[bj % 2], out_ref.at[bj], sout.at[bj % 2])
        o.start(priority=bj % 2)
        outs[bj] = o

    for b in range(B):
        s = b % 2
        if b >= 2:
            outs[b - 2].wait()
        ins[b] = []
        for k in range(TOPK):
            _, _, tk = frames[b * TOPK + k]
            c = pltpu.make_async_copy(value_ref.at[b, tk], bufs[s].at[k],
                                      sin.at[s * TOPK + k])
            c.start(priority=(b * TOPK + k) % 2)
            ins[b].append(c)
        if b >= 1:
            start_out(b - 1)
    start_out(B - 1)
    outs[B - 2].wait()
    outs[B - 1].wait()


def kernel(query, key, value, in_proj_w, in_proj_b, out_proj_w, out_proj_b,
           lin1_w, lin1_b, lin2_w, lin2_b, ln_w, ln_b):
    wq = in_proj_w[:D]
    wk = in_proj_w[D:2 * D]
    bq = in_proj_b[:D].reshape(1, D)
    bk = in_proj_b[D:2 * D].reshape(1, D)
    return pl.pallas_call(
        _fused_kernel,
        in_specs=[
            pl.BlockSpec(memory_space=pltpu.VMEM),
            pl.BlockSpec(memory_space=pltpu.VMEM),
            pl.BlockSpec(memory_space=pltpu.VMEM),
            pl.BlockSpec(memory_space=pltpu.VMEM),
            pl.BlockSpec(memory_space=pltpu.VMEM),
            pl.BlockSpec(memory_space=pltpu.VMEM),
            pl.BlockSpec(memory_space=pltpu.HBM),
        ],
        out_specs=pl.BlockSpec(memory_space=pltpu.HBM),
        out_shape=jax.ShapeDtypeStruct((B, TOPK, N, D), jnp.float32),
        scratch_shapes=[
            [pltpu.VMEM((TOPK, N, D), jnp.float32) for _ in range(2)],
            pltpu.SemaphoreType.DMA((2 * TOPK,)),
            pltpu.SemaphoreType.DMA((2,)),
        ],
    )(query, key, wq, wk, bq, bk, value)
